# Initial kernel scaffold; baseline (speedup 1.0000x reference)
import jax
import jax.numpy as jnp
from jax.experimental import pallas as pl


def kernel(indices, W_embedding):
    raise NotImplementedError("write your pallas kernel here")


# speedup vs baseline: 1.1126x; 1.1126x over previous
import functools

import jax
import jax.numpy as jnp
from jax import lax
from jax.experimental import pallas as pl
from jax.experimental.pallas import tpu as pltpu
from jax.experimental.pallas import tpu_sc as plsc


def _sc_workers():
    try:
        info = plsc.get_sparse_core_info()
        return info.num_cores, info.num_subcores
    except Exception:
        return 2, 16


@functools.partial(jax.jit, static_argnames=("n_per_w", "chunk"))
def _gather_rows(idx, table, *, n_per_w, chunk):
    nc, ns = _sc_workers()
    nw = nc * ns
    n_chunks = n_per_w // chunk
    d = table.shape[1]
    n_total = nw * n_per_w

    mesh = plsc.VectorSubcoreMesh(core_axis_name="c", subcore_axis_name="s")

    @functools.partial(
        pl.kernel,
        out_type=jax.ShapeDtypeStruct((n_total, d), table.dtype),
        mesh=mesh,
        scratch_types=[
            pltpu.VMEM((n_per_w,), jnp.int32),
            pltpu.VMEM((2, chunk, d), table.dtype),
            pltpu.SemaphoreType.DMA,
            pltpu.SemaphoreType.DMA,
            pltpu.SemaphoreType.DMA,
            pltpu.SemaphoreType.DMA,
        ],
        compiler_params=pltpu.CompilerParams(use_tc_tiling_on_sc=False),
    )
    def body(idx_hbm, table_hbm, out_hbm, idx_v, rows_v, g0, g1, o0, o1):
        wid = lax.axis_index("s") * nc + lax.axis_index("c")
        base = wid * n_per_w

        pltpu.sync_copy(idx_hbm.at[wid], idx_v)

        def fire_gather(g, buf, sem):
            pltpu.async_copy(
                table_hbm.at[idx_v.at[pl.ds(g * chunk, chunk)]],
                rows_v.at[buf],
                sem,
            )

        def drain_gather(buf, sem):
            pltpu.make_async_copy(
                table_hbm.at[idx_v.at[pl.ds(0, chunk)]], rows_v.at[buf], sem
            ).wait()

        def fire_store(g, buf, sem):
            pltpu.async_copy(
                rows_v.at[buf], out_hbm.at[pl.ds(base + g * chunk, chunk)], sem
            )

        def drain_store(g, buf, sem):
            pltpu.make_async_copy(
                rows_v.at[buf], out_hbm.at[pl.ds(base + g * chunk, chunk)], sem
            ).wait()

        fire_gather(0, 0, g0)
        fire_gather(1, 1, g1)
        drain_gather(0, g0)
        fire_store(0, 0, o0)

        @pl.loop(1, n_chunks // 2)
        def _(t):
            ge = 2 * t
            drain_store(ge - 2, 0, o0)
            fire_gather(ge, 0, g0)
            drain_gather(1, g1)
            fire_store(ge - 1, 1, o1)

            go = 2 * t + 1
            drain_store(go - 2, 1, o1)
            fire_gather(go, 1, g1)
            drain_gather(0, g0)
            fire_store(go - 1, 0, o0)

        drain_gather(1, g1)
        fire_store(n_chunks - 1, 1, o1)
        drain_store(n_chunks - 2, 0, o0)
        drain_store(n_chunks - 1, 1, o1)

    return body(idx, table)


def kernel(indices, W_embedding):
    b, h = indices.shape
    v, d = W_embedding.shape
    n = b * h
    nc, ns = _sc_workers()
    nw = nc * ns

    chunk = 1280
    n_per_w = n // nw
    assert n % nw == 0 and n_per_w % chunk == 0 and (n_per_w // chunk) % 2 == 0

    idx = indices.reshape(nw, n_per_w).astype(jnp.int32)
    out = _gather_rows(idx, W_embedding, n_per_w=n_per_w, chunk=chunk)
    return out.reshape(b, h, d)


# speedup vs baseline: 1.6049x; 1.4426x over previous
import functools

import jax
import jax.numpy as jnp
from jax import lax
from jax.experimental import pallas as pl
from jax.experimental.pallas import tpu as pltpu
from jax.experimental.pallas import tpu_sc as plsc


def _sc_workers():
    try:
        info = plsc.get_sparse_core_info()
        return info.num_cores, info.num_subcores
    except Exception:
        return 2, 16


@functools.partial(jax.jit, static_argnames=("h_dim", "b_dim"))
def _gather_t(idx, table, *, h_dim, b_dim):
    nc, ns = _sc_workers()
    nw = nc * ns
    d = table.shape[1]
    n_per_w = idx.shape[1]

    cb = 5
    chunk = cb * 128
    n_chunks = n_per_w // chunk
    blocks_per_w = n_per_w // 128
    n_cb = b_dim // 128
    out_rows = h_dim * (d // 8) * n_cb * 8
    assert n_chunks % 2 == 0 and d % 8 == 0

    mesh = plsc.VectorSubcoreMesh(core_axis_name="c", subcore_axis_name="s")

    @functools.partial(
        pl.kernel,
        out_type=jax.ShapeDtypeStruct((out_rows, 128), table.dtype),
        mesh=mesh,
        scratch_types=[
            pltpu.VMEM((n_per_w,), jnp.int32),
            pltpu.VMEM((2, chunk, d), table.dtype),
            pltpu.VMEM((cb * d, 128), table.dtype),
            pltpu.SemaphoreType.DMA,
            pltpu.SemaphoreType.DMA,
            pltpu.SemaphoreType.DMA,
        ],
        compiler_params=pltpu.CompilerParams(
            needs_layout_passes=False, use_tc_tiling_on_sc=False
        ),
    )
    def body(idx_hbm, table_hbm, out_hbm, idx_v, rows_v, trans_v, g0, g1, st):
        wid = lax.axis_index("s") * nc + lax.axis_index("c")
        base_blk = wid * blocks_per_w

        pltpu.sync_copy(idx_hbm.at[wid], idx_v)

        iota = jax.lax.iota(jnp.int32, 16)

        def fire_gather(g, buf, sem):
            pltpu.async_copy(
                table_hbm.at[idx_v.at[pl.ds(g * chunk, chunk)]],
                rows_v.at[buf],
                sem,
            )

        def drain_gather(buf, sem):
            pltpu.make_async_copy(
                table_hbm.at[idx_v.at[pl.ds(0, chunk)]], rows_v.at[buf], sem
            ).wait()

        def permute_chunk(buf):
            rows = rows_v.at[buf]

            @pl.loop(0, cb * d)
            def _(t):
                blk = t // d
                jc = t - blk * d
                col = jnp.full((16,), jc, dtype=jnp.int32)
                rb = blk * 128
                for k in range(8):
                    row = rb + k * 16 + iota
                    vals = plsc.load_gather(rows, [row, col])
                    trans_v[t, pl.ds(k * 16, 16)] = vals

        def fire_stores(g):
            b0 = base_blk + g * cb
            for blk in range(cb):
                bid = b0 + blk
                h = bid // n_cb
                cbk = bid - h * n_cb
                for r in range(d // 8):
                    rowb = ((h * (d // 8) + r) * n_cb + cbk) * 8
                    pltpu.async_copy(
                        trans_v.at[pl.ds(blk * d + r * 8, 8)],
                        out_hbm.at[pl.ds(rowb, 8)],
                        st,
                    )

        def drain_stores():
            for _ in range(cb * (d // 8)):
                pltpu.make_async_copy(
                    trans_v.at[pl.ds(0, 8)], out_hbm.at[pl.ds(0, 8)], st
                ).wait()

        fire_gather(0, 0, g0)
        fire_gather(1, 1, g1)
        drain_gather(0, g0)
        permute_chunk(0)
        fire_gather(2, 0, g0)
        fire_stores(0)

        @pl.loop(0, n_chunks // 2 - 1)
        def _(t):
            a = 2 * t + 1
            drain_gather(1, g1)
            drain_stores()
            permute_chunk(1)
            fire_gather(a + 2, 1, g1)
            fire_stores(a)

            b = 2 * t + 2
            drain_gather(0, g0)
            drain_stores()
            permute_chunk(0)

            @pl.when(b + 2 < n_chunks)
            def _():
                fire_gather(b + 2, 0, g0)

            fire_stores(b)

        drain_gather(1, g1)
        drain_stores()
        permute_chunk(1)
        fire_stores(n_chunks - 1)
        drain_stores()

    return body(idx, table)


def kernel(indices, W_embedding):
    b, h = indices.shape
    v, d = W_embedding.shape
    nc, ns = _sc_workers()
    nw = nc * ns
    n = b * h
    n_per_w = n // nw
    assert n % nw == 0

    idx_t = indices.T.reshape(nw, n_per_w).astype(jnp.int32)
    out5 = _gather_t(idx_t, W_embedding, h_dim=h, b_dim=b)
    out = (
        out5.reshape(h, d // 8, b // 128, 8, 128)
        .transpose(2, 4, 0, 1, 3)
        .reshape(b, h, d)
    )
    return out
